# baseline (speedup 1.0000x reference)
import functools

import jax
import jax.numpy as jnp
import numpy as np
from jax import lax
from jax.experimental import pallas as pl
from jax.experimental.pallas import tpu as pltpu
from jax.experimental.pallas import tpu_sc as plsc

N = 5000
NPAD = 5120
NCLS = 21
CHUNKS = NPAD // 16
NEG_INF = np.float32(-np.inf)


P = NPAD // 16
CH_T = P // 16


@functools.lru_cache(maxsize=None)
def _make_sc_nms():
  mesh = plsc.VectorSubcoreMesh(
      core_axis_name="c", subcore_axis_name="s", num_cores=2,
      num_subcores=16)
  return functools.partial(
      pl.kernel,
      out_type=[jax.ShapeDtypeStruct((NPAD,), jnp.float32)] * 7,
      mesh=mesh,
      scratch_types=[pltpu.VMEM((NPAD,), jnp.float32) for _ in range(5)]
      + [pltpu.VMEM((P,), jnp.float32), pltpu.VMEM((P,), jnp.float32),
         pltpu.VMEM((16,), jnp.float32), pltpu.VMEM((128,), jnp.float32),
         pltpu.VMEM((128,), jnp.float32),
         pltpu.VMEM_SHARED((256,), jnp.float32),
         pltpu.VMEM((21 * P,), jnp.float32), pltpu.VMEM((2 * P,), jnp.float32),
         pltpu.VMEM((16,), jnp.float32)],
      compiler_params=pltpu.CompilerParams(needs_layout_passes=False),
  )(_sc_nms_body)


def _sc_nms_body(conf_hbm, loc_hbm, tb_hbm,
                 kept_hbm, ms_hbm, x1_hbm, y1_hbm, x2_hbm, y2_hbm, ar_hbm,
                 x1_v, y1_v, x2_v, y2_v, ar_v,
                 msl_v, keptl_v, pub_v, rd_v, acc_v, shared_v,
                 conf_v, loc_v, tb_v):
  cid = lax.axis_index("c")
  sid = lax.axis_index("s")
  base = sid * P

  lanes = lax.iota(jnp.int32, 16)
  zero16 = jnp.zeros((16,), jnp.float32)
  izero16 = jnp.zeros((16,), jnp.int32)
  ones16 = jnp.ones((16,), jnp.float32)
  false16 = jnp.zeros((16,), jnp.bool_)
  neg16 = jnp.full((16,), NEG_INF)
  big = np.int32(1 << 30)
  bigv = jnp.full((16,), big, jnp.int32)
  lane0 = lanes == 0
  lane1 = lanes == 1
  lane2 = lanes == 2
  lane3 = lanes == 3

  pltpu.sync_copy(conf_hbm.at[pl.ds(NCLS * base, NCLS * P)], conf_v)
  pltpu.sync_copy(loc_hbm.at[pl.ds(2 * base, 2 * P)], loc_v)
  pltpu.sync_copy(tb_hbm, tb_v)
  def prep_body(j, carry):
    t0 = plsc.load_gather(tb_v, [izero16 + 8])
    t1 = plsc.load_gather(tb_v, [izero16 + 9])
    t2 = plsc.load_gather(tb_v, [izero16 + 10])
    t3 = plsc.load_gather(tb_v, [izero16 + 11])
    r16 = j * 16 + lanes
    cb = r16 * NCLS
    score = plsc.load_gather(conf_v, [cb])
    cm = score
    for c in range(1, NCLS):
      cm = jnp.maximum(cm, plsc.load_gather(conf_v, [cb + c]))
    msv = jnp.where(cm > np.float32(0.5), score, neg16)
    lx = plsc.load_gather(loc_v, [r16 * 2])
    ly = plsc.load_gather(loc_v, [r16 * 2 + 1])
    bx1 = t0 * lx
    by1 = t1 * ly
    bx2 = t2 * lx
    by2 = t3 * ly
    msl_v[pl.ds(j * 16, 16)] = msv
    gsl = pl.ds(base + j * 16, 16)
    x1_v[gsl] = bx1
    y1_v[gsl] = by1
    x2_v[gsl] = bx2
    y2_v[gsl] = by2
    ar_v[gsl] = (bx2 - bx1) * (by2 - by1)
    return carry

  lax.fori_loop(0, CH_T, prep_body, 0)
  pltpu.sync_copy(msl_v, ms_hbm.at[pl.ds(base, P)])
  pltpu.sync_copy(x1_v.at[pl.ds(base, P)], x1_hbm.at[pl.ds(base, P)])
  pltpu.sync_copy(y1_v.at[pl.ds(base, P)], y1_hbm.at[pl.ds(base, P)])
  pltpu.sync_copy(x2_v.at[pl.ds(base, P)], x2_hbm.at[pl.ds(base, P)])
  pltpu.sync_copy(y2_v.at[pl.ds(base, P)], y2_hbm.at[pl.ds(base, P)])
  pltpu.sync_copy(ar_v.at[pl.ds(base, P)], ar_hbm.at[pl.ds(base, P)])
  plsc.subcore_barrier()
  pltpu.sync_copy(x1_hbm, x1_v)
  pltpu.sync_copy(y1_hbm, y1_v)
  pltpu.sync_copy(x2_hbm, x2_v)
  pltpu.sync_copy(y2_hbm, y2_v)
  pltpu.sync_copy(ar_hbm, ar_v)

  def zero_body(j, carry):
    keptl_v[pl.ds(j * 16, 16)] = zero16
    return carry

  lax.fori_loop(0, CH_T, zero_body, 0)

  def top2_publish_read(par):
    def am(j, carry):
      rm1, ri1, rm2, ri2 = carry
      c = msl_v[pl.ds(j * 16, 16)]
      gi = base + j * 16 + lanes
      gt1 = c > rm1
      gt2 = c > rm2
      rm2n = jnp.where(gt1, rm1, jnp.where(gt2, c, rm2))
      ri2n = jnp.where(gt1, ri1, jnp.where(gt2, gi, ri2))
      rm1n = jnp.where(gt1, c, rm1)
      ri1n = jnp.where(gt1, gi, ri1)
      return rm1n, ri1n, rm2n, ri2n

    rm1, ri1, rm2, ri2 = lax.fori_loop(
        0, CH_T, am, (neg16, izero16, neg16, izero16))
    s1 = jnp.max(rm1)
    i1 = jnp.min(jnp.where(rm1 == s1, ri1, bigv))
    wl = (rm1 == s1) & (ri1 == i1)
    rm1b = jnp.where(wl, rm2, rm1)
    ri1b = jnp.where(wl, ri2, ri1)
    s2 = jnp.max(rm1b)
    i2 = jnp.min(jnp.where(rm1b == s2, ri1b, bigv))

    i1f = plsc.bitcast(jnp.full((16,), i1, jnp.int32), jnp.float32)
    i2f = plsc.bitcast(jnp.full((16,), i2, jnp.int32), jnp.float32)
    vec = jnp.where(lane0, jnp.full((16,), s1, jnp.float32),
                    jnp.where(lane1, i1f,
                              jnp.where(lane2,
                                        jnp.full((16,), s2, jnp.float32),
                                        jnp.where(lane3, i2f, zero16))))
    pub_v[...] = vec
    pltpu.sync_copy(pub_v.at[pl.ds(0, 8)],
                    shared_v.at[pl.ds(par * 128 + 8 * sid, 8)])
    plsc.subcore_barrier()
    pltpu.sync_copy(shared_v.at[pl.ds(par * 128, 128)], rd_v)
    s1s = plsc.load_gather(rd_v, [lanes * 8])
    i1s = plsc.bitcast(plsc.load_gather(rd_v, [lanes * 8 + 1]), jnp.int32)
    s2s = plsc.load_gather(rd_v, [lanes * 8 + 2])
    i2s = plsc.bitcast(plsc.load_gather(rd_v, [lanes * 8 + 3]), jnp.int32)
    return s1s, i1s, s2s, i2s

  pool0 = top2_publish_read(0)
  mg0 = jnp.max(pool0[0])

  def cond(state):
    return state[1] > NEG_INF

  def body(state):
    par, _, s1s, i1s, s2s, i2s = state

    def bcond(st):
      return jnp.logical_not(st[0])

    def bbody(st):
      (stop, steps, u1, u2, am, aI,
       ax1, ay1, ax2, ay2, aa) = st
      c1v = jnp.where(u1, s1s, neg16)
      c2v = jnp.where(u2, s2s, neg16)
      m = jnp.max(jnp.maximum(c1v, c2v))
      candv = jnp.minimum(jnp.where(u1 & (s1s == m), i1s, bigv),
                          jnp.where(u2 & (s2s == m), i2s, bigv))
      ci = jnp.min(candv)
      xb = jnp.max(jnp.where(u2, neg16, s2s))
      go = (m > NEG_INF) & (m > xb)

      pv = jnp.full((16,), ci, jnp.int32)
      cx1 = plsc.load_gather(x1_v, [pv])
      cy1 = plsc.load_gather(y1_v, [pv])
      cx2 = plsc.load_gather(x2_v, [pv])
      cy2 = plsc.load_gather(y2_v, [pv])
      ca = plsc.load_gather(ar_v, [pv])
      xx1 = jnp.maximum(ax1, cx1)
      yy1 = jnp.maximum(ay1, cy1)
      xx2 = jnp.minimum(ax2, cx2)
      yy2 = jnp.minimum(ay2, cy2)
      w = jnp.maximum(xx2 - xx1, np.float32(0.0))
      h = jnp.maximum(yy2 - yy1, np.float32(0.0))
      inter = w * h
      iou = inter / (aa + ca - inter + np.float32(1e-12))
      suppv = am & (iou > np.float32(0.5))
      suppb = plsc.all_reduce_population_count(suppv) > 0

      u1n = jnp.where(go, u1 & (i1s != ci), u1)
      u2n = jnp.where(go, u2 & (i2s != ci), u2)
      slot = plsc.all_reduce_ffs(jnp.logical_not(am))
      sel = (lanes == slot) & jnp.logical_not(suppb) & go
      amn = am | sel
      aIn = jnp.where(sel, pv, aI)
      ax1n = jnp.where(sel, cx1, ax1)
      ay1n = jnp.where(sel, cy1, ay1)
      ax2n = jnp.where(sel, cx2, ax2)
      ay2n = jnp.where(sel, cy2, ay2)
      aan = jnp.where(sel, ca, aa)
      steps2 = steps + 1
      stop2 = jnp.logical_not(go) | (steps2 >= 16)
      return (stop2, steps2, u1n, u2n, amn, aIn,
              ax1n, ay1n, ax2n, ay2n, aan)

    init = (False, np.int32(0),
            jnp.ones((16,), jnp.bool_), jnp.ones((16,), jnp.bool_),
            false16, izero16, zero16, zero16, zero16, zero16, zero16)
    (_, _, _, _, am, aI, ax1, ay1, ax2, ay2, aa) = lax.while_loop(
        bcond, bbody, init)

    offv = aI - base
    wm = am & (offv >= 0) & (offv < P)
    offc = jnp.clip(offv, 0, P - 1)
    plsc.store_scatter(keptl_v, [offc], ones16, mask=wm)
    plsc.store_scatter(msl_v, [offc], neg16, mask=wm)

    acc_v[pl.ds(0, 16)] = ax1
    acc_v[pl.ds(16, 16)] = ay1
    acc_v[pl.ds(32, 16)] = ax2
    acc_v[pl.ds(48, 16)] = ay2
    acc_v[pl.ds(64, 16)] = aa
    cnt = jnp.max(plsc.all_reduce_population_count(am))

    def per_pivot(k, carry):
      kv = jnp.full((16,), k, jnp.int32)
      px1 = plsc.load_gather(acc_v, [kv])
      py1 = plsc.load_gather(acc_v, [kv + 16])
      px2 = plsc.load_gather(acc_v, [kv + 32])
      py2 = plsc.load_gather(acc_v, [kv + 48])
      pa = plsc.load_gather(acc_v, [kv + 64])

      def sweep(j, c2):
        sll = pl.ds(j * 16, 16)
        gb = base + j * 16
        al = msl_v[sll]
        bx1 = x1_v[pl.ds(gb, 16)]
        by1 = y1_v[pl.ds(gb, 16)]
        bx2 = x2_v[pl.ds(gb, 16)]
        by2 = y2_v[pl.ds(gb, 16)]
        ba = ar_v[pl.ds(gb, 16)]
        xx1 = jnp.maximum(px1, bx1)
        yy1 = jnp.maximum(py1, by1)
        xx2 = jnp.minimum(px2, bx2)
        yy2 = jnp.minimum(py2, by2)
        w = jnp.maximum(xx2 - xx1, np.float32(0.0))
        h = jnp.maximum(yy2 - yy1, np.float32(0.0))
        inter = w * h
        iou = inter / (pa + ba - inter + np.float32(1e-12))
        msl_v[sll] = jnp.where(iou > np.float32(0.5), neg16, al)
        return c2

      lax.fori_loop(0, CH_T, sweep, 0)
      return carry

    lax.fori_loop(0, cnt, per_pivot, 0)

    par2 = 1 - par
    s1s2, i1s2, s2s2, i2s2 = top2_publish_read(par2)
    mg = jnp.max(s1s2)
    return par2, mg, s1s2, i1s2, s2s2, i2s2

  lax.while_loop(cond, body, (np.int32(0), mg0) + pool0)

  @pl.when(cid == 0)
  def _():
    pltpu.sync_copy(keptl_v, kept_hbm.at[pl.ds(base, P)])


def _loss_body(msr_ref, keptr_ref, kept_ref, conf_ref,
               x1_ref, y1_ref, x2_ref, y2_ref, tb_ref, out_ref):
  maskr = (msr_ref[...] > NEG_INF).astype(jnp.float32)
  keptr = keptr_ref[...]
  kept = kept_ref[...]
  conf = conf_ref[...]

  rows = maskr.shape[0]
  cols = maskr.shape[1]
  io_r = lax.broadcasted_iota(jnp.int32, (cols, cols), 0)
  io_c = lax.broadcasted_iota(jnp.int32, (cols, cols), 1)
  upper = (io_r <= io_c).astype(jnp.float32)
  within = lax.dot(maskr, upper,
                   preferred_element_type=jnp.float32)
  rowsum = within[:, cols - 1:cols]
  lo_r = lax.broadcasted_iota(jnp.int32, (rows, rows), 0)
  lo_c = lax.broadcasted_iota(jnp.int32, (rows, rows), 1)
  lower = (lo_c < lo_r).astype(jnp.float32)
  offs = lax.dot(lower, rowsum,
                 preferred_element_type=jnp.float32)
  ranks = within + offs - np.float32(1.0)
  num_positives = jnp.sum(keptr * ranks)

  keptb = kept > np.float32(0.5)
  mc = jnp.where(keptb, conf, NEG_INF)
  maxv = jnp.max(mc, axis=1, keepdims=True)
  colio = lax.broadcasted_iota(jnp.int32, (NCLS, NPAD), 1)
  idx = jnp.min(jnp.where(mc == maxv, colio, np.int32(1 << 30)),
                axis=1, keepdims=True)
  onehot = (colio == idx).astype(jnp.float32)

  mlx1 = jnp.sum(onehot * x1_ref[...], axis=1, keepdims=True)
  mly1 = jnp.sum(onehot * y1_ref[...], axis=1, keepdims=True)
  mlx2 = jnp.sum(onehot * x2_ref[...], axis=1, keepdims=True)
  mly2 = jnp.sum(onehot * y2_ref[...], axis=1, keepdims=True)

  def smooth_l1(d):
    ad = jnp.abs(d)
    return jnp.where(ad < np.float32(1.0),
                     np.float32(0.5) * d * d,
                     ad - np.float32(0.5))

  t0 = tb_ref[0:1, 0:1]
  t1 = tb_ref[0:1, 1:2]
  t2 = tb_ref[0:1, 2:3]
  t3 = tb_ref[0:1, 3:4]
  loc_loss = (jnp.sum(smooth_l1(mlx1 - t0)) +
              jnp.sum(smooth_l1(mly1 - t1)) +
              jnp.sum(smooth_l1(mlx2 - t2)) +
              jnp.sum(smooth_l1(mly2 - t3)))

  total = loc_loss / num_positives
  any_valid = jnp.max(maskr) > np.float32(0.0)
  has_keep = jnp.max(keptr) > np.float32(0.0)
  res = jnp.where(any_valid & has_keep, total, np.float32(0.001))
  out_ref[...] = jnp.full((1, 1), res, jnp.float32)


_loss = pl.pallas_call(
    _loss_body,
    out_shape=jax.ShapeDtypeStruct((1, 1), jnp.float32),
)


def kernel(loc, conf, target_boxes, target_labels):
  del target_labels
  confp = jnp.pad(conf.T, ((0, 0), (0, NPAD - N)))
  conf_flat = jnp.pad(conf, ((0, NPAD - N), (0, 0))).reshape(NPAD * NCLS)
  loc_flat = jnp.pad(loc[0], ((0, NPAD - N), (0, 0))).reshape(NPAD * 2)
  tb16 = jnp.pad(target_boxes.reshape(4), (8, 4))
  tb4 = target_boxes.reshape(1, 4)

  kept, ms, x1, y1, x2, y2, ar7 = _make_sc_nms()(conf_flat, loc_flat, tb16)

  out = _loss(ms.reshape(40, 128), kept.reshape(40, 128),
              kept.reshape(1, NPAD), confp,
              x1.reshape(1, NPAD), y1.reshape(1, NPAD),
              x2.reshape(1, NPAD), y2.reshape(1, NPAD), tb4)
  return out[0, 0]


# speedup vs baseline: 1.5379x; 1.5379x over previous
import functools

import jax
import jax.numpy as jnp
import numpy as np
from jax import lax
from jax.experimental import pallas as pl
from jax.experimental.pallas import tpu as pltpu
from jax.experimental.pallas import tpu_sc as plsc

N = 5000
NPAD = 5120
NCLS = 21
CHUNKS = NPAD // 16
NEG_INF = np.float32(-np.inf)


def _prep_body(conf_ref, lx_ref, ly_ref, tb_ref,
               ms_ref, x1_ref, y1_ref, x2_ref, y2_ref, ar_ref, mk_ref):
  conf = conf_ref[...]
  lx = lx_ref[...]
  ly = ly_ref[...]
  t0 = tb_ref[0:1, 0:1]
  t1 = tb_ref[0:1, 1:2]
  t2 = tb_ref[0:1, 2:3]
  t3 = tb_ref[0:1, 3:4]
  cmax = jnp.max(conf, axis=0, keepdims=True)
  mask = cmax > np.float32(0.5)
  score = conf[0:1, :]
  ms_ref[...] = jnp.where(mask, score, NEG_INF)
  x1 = t0 * lx
  y1 = t1 * ly
  x2 = t2 * lx
  y2 = t3 * ly
  x1_ref[...] = x1
  y1_ref[...] = y1
  x2_ref[...] = x2
  y2_ref[...] = y2
  ar_ref[...] = (x2 - x1) * (y2 - y1)
  mk_ref[...] = mask.astype(jnp.float32)


_prep = pl.pallas_call(
    _prep_body,
    out_shape=[jax.ShapeDtypeStruct((1, NPAD), jnp.float32)] * 7,
)


P = NPAD // 16
CH_T = P // 16


@functools.lru_cache(maxsize=None)
def _make_sc_nms():
  mesh = plsc.VectorSubcoreMesh(
      core_axis_name="c", subcore_axis_name="s", num_cores=2,
      num_subcores=16)
  return functools.partial(
      pl.kernel,
      out_type=jax.ShapeDtypeStruct((NPAD,), jnp.float32),
      mesh=mesh,
      scratch_types=[pltpu.VMEM((NPAD,), jnp.float32) for _ in range(5)]
      + [pltpu.VMEM((P,), jnp.float32), pltpu.VMEM((P,), jnp.float32),
         pltpu.VMEM((16,), jnp.float32), pltpu.VMEM((128,), jnp.float32),
         pltpu.VMEM((128,), jnp.float32),
         pltpu.VMEM_SHARED((256,), jnp.float32),
         pltpu.VMEM((P + 16,), jnp.int32)]
      + [pltpu.VMEM((P + 16,), jnp.float32) for _ in range(6)],
      compiler_params=pltpu.CompilerParams(needs_layout_passes=False),
  )(_sc_nms_body)


def _sc_nms_body(ms_hbm, x1_hbm, y1_hbm, x2_hbm, y2_hbm, ar_hbm, kept_hbm,
                 x1_v, y1_v, x2_v, y2_v, ar_v,
                 msl_v, keptl_v, pub_v, rd_v, acc_v, shared_v,
                 gx_p, sc_p, x1_p, y1_p, x2_p, y2_p, ar_p):
  cid = lax.axis_index("c")
  sid = lax.axis_index("s")
  base = sid * P

  lanes = lax.iota(jnp.int32, 16)
  zero16 = jnp.zeros((16,), jnp.float32)
  izero16 = jnp.zeros((16,), jnp.int32)
  ones16 = jnp.ones((16,), jnp.float32)
  false16 = jnp.zeros((16,), jnp.bool_)
  neg16 = jnp.full((16,), NEG_INF)
  big = np.int32(1 << 30)
  bigv = jnp.full((16,), big, jnp.int32)
  lane0 = lanes == 0
  lane1 = lanes == 1
  lane2 = lanes == 2
  lane3 = lanes == 3

  pltpu.sync_copy(x1_hbm, x1_v)
  pltpu.sync_copy(y1_hbm, y1_v)
  pltpu.sync_copy(x2_hbm, x2_v)
  pltpu.sync_copy(y2_hbm, y2_v)
  pltpu.sync_copy(ar_hbm, ar_v)
  pltpu.sync_copy(ms_hbm.at[pl.ds(base, P)], msl_v)

  def zero_body(j, carry):
    keptl_v[pl.ds(j * 16, 16)] = zero16
    return carry

  lax.fori_loop(0, CH_T, zero_body, 0)

  def extract_top2(rm1, ri1, rm2, ri2):
    s1 = jnp.max(rm1)
    i1 = jnp.min(jnp.where(rm1 == s1, ri1, bigv))
    wl = (rm1 == s1) & (ri1 == i1)
    rm1b = jnp.where(wl, rm2, rm1)
    ri1b = jnp.where(wl, ri2, ri1)
    s2 = jnp.max(rm1b)
    i2 = jnp.min(jnp.where(rm1b == s2, ri1b, bigv))
    return s1, i1, s2, i2

  def top2_update(carry, c, gi):
    rm1, ri1, rm2, ri2 = carry
    gt1 = c > rm1
    gt2 = c > rm2
    rm2n = jnp.where(gt1, rm1, jnp.where(gt2, c, rm2))
    ri2n = jnp.where(gt1, ri1, jnp.where(gt2, gi, ri2))
    rm1n = jnp.where(gt1, c, rm1)
    ri1n = jnp.where(gt1, gi, ri1)
    return rm1n, ri1n, rm2n, ri2n

  def publish_read(par, s1, i1, s2, i2):
    i1f = plsc.bitcast(jnp.full((16,), i1, jnp.int32), jnp.float32)
    i2f = plsc.bitcast(jnp.full((16,), i2, jnp.int32), jnp.float32)
    vec = jnp.where(lane0, jnp.full((16,), s1, jnp.float32),
                    jnp.where(lane1, i1f,
                              jnp.where(lane2,
                                        jnp.full((16,), s2, jnp.float32),
                                        jnp.where(lane3, i2f, zero16))))
    pub_v[...] = vec
    pltpu.sync_copy(pub_v.at[pl.ds(0, 8)],
                    shared_v.at[pl.ds(par * 128 + 8 * sid, 8)])
    plsc.subcore_barrier()
    pltpu.sync_copy(shared_v.at[pl.ds(par * 128, 128)], rd_v)
    s1s = plsc.load_gather(rd_v, [lanes * 8])
    i1s = plsc.bitcast(plsc.load_gather(rd_v, [lanes * 8 + 1]), jnp.int32)
    s2s = plsc.load_gather(rd_v, [lanes * 8 + 2])
    i2s = plsc.bitcast(plsc.load_gather(rd_v, [lanes * 8 + 3]), jnp.int32)
    return s1s, i1s, s2s, i2s

  def init_pack(j, carry):
    woff, rm1, ri1, rm2, ri2 = carry
    al = msl_v[pl.ds(j * 16, 16)]
    gb = base + j * 16
    gx = gb + lanes
    keep = al > NEG_INF
    dst = pl.ds(woff, 16)
    plsc.store_compressed(gx_p.at[dst], gx, mask=keep)
    plsc.store_compressed(sc_p.at[dst], al, mask=keep)
    plsc.store_compressed(x1_p.at[dst], x1_v[pl.ds(gb, 16)], mask=keep)
    plsc.store_compressed(y1_p.at[dst], y1_v[pl.ds(gb, 16)], mask=keep)
    plsc.store_compressed(x2_p.at[dst], x2_v[pl.ds(gb, 16)], mask=keep)
    plsc.store_compressed(y2_p.at[dst], y2_v[pl.ds(gb, 16)], mask=keep)
    plsc.store_compressed(ar_p.at[dst], ar_v[pl.ds(gb, 16)], mask=keep)
    cntc = jnp.max(plsc.all_reduce_population_count(keep))
    carry2 = top2_update((rm1, ri1, rm2, ri2),
                         jnp.where(keep, al, neg16), gx)
    return (woff + cntc,) + carry2

  woff, rm1, ri1, rm2, ri2 = lax.fori_loop(
      0, CH_T, init_pack,
      (np.int32(0), neg16, izero16, neg16, izero16))
  acnt0 = woff
  s1, i1, s2, i2 = extract_top2(rm1, ri1, rm2, ri2)
  pool0 = publish_read(0, s1, i1, s2, i2)
  mg0 = jnp.max(pool0[0])

  def cond(state):
    return state[1] > NEG_INF

  def body(state):
    par, _, acnt, s1s, i1s, s2s, i2s = state

    def bcond(st):
      return jnp.logical_not(st[0])

    def bbody(st):
      (stop, steps, u1, u2, am, aI,
       ax1, ay1, ax2, ay2, aa) = st
      c1v = jnp.where(u1, s1s, neg16)
      c2v = jnp.where(u2, s2s, neg16)
      m = jnp.max(jnp.maximum(c1v, c2v))
      candv = jnp.minimum(jnp.where(u1 & (s1s == m), i1s, bigv),
                          jnp.where(u2 & (s2s == m), i2s, bigv))
      ci = jnp.min(candv)
      xb = jnp.max(jnp.where(u2, neg16, s2s))
      go = (m > NEG_INF) & (m > xb)

      pv = jnp.full((16,), ci, jnp.int32)
      cx1 = plsc.load_gather(x1_v, [pv])
      cy1 = plsc.load_gather(y1_v, [pv])
      cx2 = plsc.load_gather(x2_v, [pv])
      cy2 = plsc.load_gather(y2_v, [pv])
      ca = plsc.load_gather(ar_v, [pv])
      xx1 = jnp.maximum(ax1, cx1)
      yy1 = jnp.maximum(ay1, cy1)
      xx2 = jnp.minimum(ax2, cx2)
      yy2 = jnp.minimum(ay2, cy2)
      w = jnp.maximum(xx2 - xx1, np.float32(0.0))
      h = jnp.maximum(yy2 - yy1, np.float32(0.0))
      inter = w * h
      iou = inter / (aa + ca - inter + np.float32(1e-12))
      suppv = am & (iou > np.float32(0.5))
      suppb = plsc.all_reduce_population_count(suppv) > 0

      u1n = jnp.where(go, u1 & (i1s != ci), u1)
      u2n = jnp.where(go, u2 & (i2s != ci), u2)
      slot = plsc.all_reduce_ffs(jnp.logical_not(am))
      sel = (lanes == slot) & jnp.logical_not(suppb) & go
      amn = am | sel
      aIn = jnp.where(sel, pv, aI)
      ax1n = jnp.where(sel, cx1, ax1)
      ay1n = jnp.where(sel, cy1, ay1)
      ax2n = jnp.where(sel, cx2, ax2)
      ay2n = jnp.where(sel, cy2, ay2)
      aan = jnp.where(sel, ca, aa)
      steps2 = steps + 1
      stop2 = jnp.logical_not(go) | (steps2 >= 16)
      return (stop2, steps2, u1n, u2n, amn, aIn,
              ax1n, ay1n, ax2n, ay2n, aan)

    init = (False, np.int32(0),
            jnp.ones((16,), jnp.bool_), jnp.ones((16,), jnp.bool_),
            false16, izero16, zero16, zero16, zero16, zero16, zero16)
    (_, _, _, _, am, aI, ax1, ay1, ax2, ay2, aa) = lax.while_loop(
        bcond, bbody, init)

    offv = aI - base
    wm = am & (offv >= 0) & (offv < P)
    offc = jnp.clip(offv, 0, P - 1)
    plsc.store_scatter(keptl_v, [offc], ones16, mask=wm)

    acc_v[pl.ds(0, 16)] = ax1
    acc_v[pl.ds(16, 16)] = ay1
    acc_v[pl.ds(32, 16)] = ax2
    acc_v[pl.ds(48, 16)] = ay2
    acc_v[pl.ds(64, 16)] = aa
    acc_v[pl.ds(80, 16)] = plsc.bitcast(aI, jnp.float32)
    cnt = jnp.max(plsc.all_reduce_population_count(am))
    chunks = (acnt + 15) // 16

    def per_pivot(k, carry):
      kv = jnp.full((16,), k, jnp.int32)
      px1 = plsc.load_gather(acc_v, [kv])
      py1 = plsc.load_gather(acc_v, [kv + 16])
      px2 = plsc.load_gather(acc_v, [kv + 32])
      py2 = plsc.load_gather(acc_v, [kv + 48])
      pa = plsc.load_gather(acc_v, [kv + 64])
      pgi = plsc.bitcast(plsc.load_gather(acc_v, [kv + 80]), jnp.int32)

      def sweep(j, c2):
        sll = pl.ds(j * 16, 16)
        al = sc_p[sll]
        bx1 = x1_p[sll]
        by1 = y1_p[sll]
        bx2 = x2_p[sll]
        by2 = y2_p[sll]
        ba = ar_p[sll]
        bgx = gx_p[sll]
        xx1 = jnp.maximum(px1, bx1)
        yy1 = jnp.maximum(py1, by1)
        xx2 = jnp.minimum(px2, bx2)
        yy2 = jnp.minimum(py2, by2)
        w = jnp.maximum(xx2 - xx1, np.float32(0.0))
        h = jnp.maximum(yy2 - yy1, np.float32(0.0))
        inter = w * h
        iou = inter / (pa + ba - inter + np.float32(1e-12))
        kill = (iou > np.float32(0.5)) | (bgx == pgi)
        sc_p[sll] = jnp.where(kill, neg16, al)
        return c2

      lax.fori_loop(0, chunks, sweep, 0)
      return carry

    lax.fori_loop(0, cnt, per_pivot, 0)

    def cpk(j, carry):
      woff, rm1, ri1, rm2, ri2 = carry
      sll = pl.ds(j * 16, 16)
      al = sc_p[sll]
      gx = gx_p[sll]
      bx1 = x1_p[sll]
      by1 = y1_p[sll]
      bx2 = x2_p[sll]
      by2 = y2_p[sll]
      ba = ar_p[sll]
      valid = (j * 16 + lanes) < acnt
      keep = (al > NEG_INF) & valid
      dst = pl.ds(woff, 16)
      plsc.store_compressed(gx_p.at[dst], gx, mask=keep)
      plsc.store_compressed(sc_p.at[dst], al, mask=keep)
      plsc.store_compressed(x1_p.at[dst], bx1, mask=keep)
      plsc.store_compressed(y1_p.at[dst], by1, mask=keep)
      plsc.store_compressed(x2_p.at[dst], bx2, mask=keep)
      plsc.store_compressed(y2_p.at[dst], by2, mask=keep)
      plsc.store_compressed(ar_p.at[dst], ba, mask=keep)
      cntc = jnp.max(plsc.all_reduce_population_count(keep))
      carry2 = top2_update((rm1, ri1, rm2, ri2),
                           jnp.where(keep, al, neg16), gx)
      return (woff + cntc,) + carry2

    woff, rm1, ri1, rm2, ri2 = lax.fori_loop(
        0, chunks, cpk, (np.int32(0), neg16, izero16, neg16, izero16))
    acnt2 = woff
    s1, i1, s2, i2 = extract_top2(rm1, ri1, rm2, ri2)

    par2 = 1 - par
    s1s2, i1s2, s2s2, i2s2 = publish_read(par2, s1, i1, s2, i2)
    mg = jnp.max(s1s2)
    return par2, mg, acnt2, s1s2, i1s2, s2s2, i2s2

  lax.while_loop(cond, body, (np.int32(0), mg0, acnt0) + pool0)


  @pl.when(cid == 0)
  def _():
    pltpu.sync_copy(keptl_v, kept_hbm.at[pl.ds(base, P)])


def _loss_body(maskr_ref, keptr_ref, kept_ref, conf_ref,
               x1_ref, y1_ref, x2_ref, y2_ref, tb_ref, out_ref):
  maskr = maskr_ref[...]
  keptr = keptr_ref[...]
  kept = kept_ref[...]
  conf = conf_ref[...]

  rows = maskr.shape[0]
  cols = maskr.shape[1]
  io_r = lax.broadcasted_iota(jnp.int32, (cols, cols), 0)
  io_c = lax.broadcasted_iota(jnp.int32, (cols, cols), 1)
  upper = (io_r <= io_c).astype(jnp.float32)
  within = lax.dot(maskr, upper,
                   preferred_element_type=jnp.float32)
  rowsum = within[:, cols - 1:cols]
  lo_r = lax.broadcasted_iota(jnp.int32, (rows, rows), 0)
  lo_c = lax.broadcasted_iota(jnp.int32, (rows, rows), 1)
  lower = (lo_c < lo_r).astype(jnp.float32)
  offs = lax.dot(lower, rowsum,
                 preferred_element_type=jnp.float32)
  ranks = within + offs - np.float32(1.0)
  num_positives = jnp.sum(keptr * ranks)

  keptb = kept > np.float32(0.5)
  mc = jnp.where(keptb, conf, NEG_INF)
  maxv = jnp.max(mc, axis=1, keepdims=True)
  colio = lax.broadcasted_iota(jnp.int32, (NCLS, NPAD), 1)
  idx = jnp.min(jnp.where(mc == maxv, colio, np.int32(1 << 30)),
                axis=1, keepdims=True)
  onehot = (colio == idx).astype(jnp.float32)

  mlx1 = jnp.sum(onehot * x1_ref[...], axis=1, keepdims=True)
  mly1 = jnp.sum(onehot * y1_ref[...], axis=1, keepdims=True)
  mlx2 = jnp.sum(onehot * x2_ref[...], axis=1, keepdims=True)
  mly2 = jnp.sum(onehot * y2_ref[...], axis=1, keepdims=True)

  def smooth_l1(d):
    ad = jnp.abs(d)
    return jnp.where(ad < np.float32(1.0),
                     np.float32(0.5) * d * d,
                     ad - np.float32(0.5))

  t0 = tb_ref[0:1, 0:1]
  t1 = tb_ref[0:1, 1:2]
  t2 = tb_ref[0:1, 2:3]
  t3 = tb_ref[0:1, 3:4]
  loc_loss = (jnp.sum(smooth_l1(mlx1 - t0)) +
              jnp.sum(smooth_l1(mly1 - t1)) +
              jnp.sum(smooth_l1(mlx2 - t2)) +
              jnp.sum(smooth_l1(mly2 - t3)))

  total = loc_loss / num_positives
  any_valid = jnp.max(maskr) > np.float32(0.0)
  has_keep = jnp.max(keptr) > np.float32(0.0)
  res = jnp.where(any_valid & has_keep, total, np.float32(0.001))
  out_ref[...] = jnp.full((1, 1), res, jnp.float32)


_loss = pl.pallas_call(
    _loss_body,
    out_shape=jax.ShapeDtypeStruct((1, 1), jnp.float32),
)


def kernel(loc, conf, target_boxes, target_labels):
  del target_labels
  confp = jnp.pad(conf.T, ((0, 0), (0, NPAD - N)))
  lxp = jnp.pad(loc[0, :, 0], (0, NPAD - N)).reshape(1, NPAD)
  lyp = jnp.pad(loc[0, :, 1], (0, NPAD - N)).reshape(1, NPAD)
  tb4 = target_boxes.reshape(1, 4)

  ms, x1, y1, x2, y2, ar, mk = _prep(confp, lxp, lyp, tb4)

  kept = _make_sc_nms()(ms.reshape(NPAD), x1.reshape(NPAD), y1.reshape(NPAD),
                        x2.reshape(NPAD), y2.reshape(NPAD), ar.reshape(NPAD))

  out = _loss(mk.reshape(40, 128), kept.reshape(40, 128),
              kept.reshape(1, NPAD), confp, x1, y1, x2, y2, tb4)
  return out[0, 0]
